# baseline (speedup 1.0000x reference)
import itertools
import functools

import numpy as np
import jax
import jax.numpy as jnp
from jax.experimental import pallas as pl

_RES = 128
_SIG = 10.0
_KZ = 40
_KXY = 72


def _freq_vals():
    kz = np.zeros(_KZ, np.float64)
    kz[:33] = np.arange(33)
    kz_mask = np.zeros(_KZ, np.float64)
    kz_mask[:33] = 1.0
    kxy = np.zeros(_KXY, np.float64)
    kxy[:65] = np.arange(-32, 33)
    kxy_mask = np.zeros(_KXY, np.float64)
    kxy_mask[:65] = 1.0
    return kz, kz_mask, kxy, kxy_mask


_KZV, _KZM, _KXYV, _KXYM = _freq_vals()


def _fwd_mats():
    n = np.arange(_RES, dtype=np.float64)
    az = 2.0 * np.pi * np.outer(n, _KZV) / _RES
    wz_c = (np.cos(az) * _KZM).astype(np.float32)
    wz_s = (np.sin(az) * _KZM).astype(np.float32)
    axy = 2.0 * np.pi * np.outer(n, _KXYV) / _RES
    wxy_c = (np.cos(axy) * _KXYM).astype(np.float32)
    wxy_s = (np.sin(axy) * _KXYM).astype(np.float32)
    return wz_c, wz_s, wxy_c, wxy_s


def _inv_mats():
    n = np.arange(_RES, dtype=np.float64)
    axy = 2.0 * np.pi * np.outer(_KXYV, n) / _RES
    iwxy_c = (np.cos(axy) * _KXYM[:, None] / _RES).astype(np.float32)
    iwxy_s = (-np.sin(axy) * _KXYM[:, None] / _RES).astype(np.float32)
    az = 2.0 * np.pi * np.outer(_KZV, n) / _RES
    cz = np.where(_KZV == 0, 1.0, 2.0) * _KZM
    iwz_c = (np.cos(az) * cz[:, None] / _RES).astype(np.float32)
    iwz_s = (np.sin(az) * cz[:, None] / _RES).astype(np.float32)
    return iwxy_c, iwxy_s, iwz_c, iwz_s


def _pointwise_w():
    kz = _KZV[:, None, None]
    ky = _KXYV[None, :, None]
    kx = _KXYV[None, None, :]
    mask = (_KZM[:, None, None] * _KXYM[None, :, None] * _KXYM[None, None, :])
    dis = np.sqrt(kx ** 2 + ky ** 2 + kz ** 2)
    G = np.exp(-0.5 * (_SIG * 2.0 * dis / _RES) ** 2)
    om = 2.0 * np.pi
    lap = -((om * kx) ** 2 + (om * ky) ** 2 + (om * kz) ** 2) + 1e-6
    shp = (_KZ, _KXY, _KXY)
    W = np.stack([np.broadcast_to(om * kx, shp), np.broadcast_to(om * ky, shp),
                  np.broadcast_to(om * kz, shp)], axis=0) * (G / lap) * mask
    return W.astype(np.float32).reshape(3, _KZ * _KXY * _KXY)


def _cplx_mm_body(re_ref, im_ref, wc_ref, ws_ref, ore_ref, oim_ref):
    r = re_ref[...]
    i = im_ref[...]
    wc = wc_ref[...]
    ws = ws_ref[...]
    ore_ref[...] = jnp.dot(r, wc, preferred_element_type=jnp.float32) + \
        jnp.dot(i, ws, preferred_element_type=jnp.float32)
    oim_ref[...] = jnp.dot(i, wc, preferred_element_type=jnp.float32) - \
        jnp.dot(r, ws, preferred_element_type=jnp.float32)


def _cplx_mm(re, im, wc, ws, bm=128):
    m, k = re.shape
    n = wc.shape[1]
    return pl.pallas_call(
        _cplx_mm_body,
        grid=(m // bm,),
        in_specs=[
            pl.BlockSpec((bm, k), lambda j: (j, 0)),
            pl.BlockSpec((bm, k), lambda j: (j, 0)),
            pl.BlockSpec((k, n), lambda j: (0, 0)),
            pl.BlockSpec((k, n), lambda j: (0, 0)),
        ],
        out_specs=[
            pl.BlockSpec((bm, n), lambda j: (j, 0)),
            pl.BlockSpec((bm, n), lambda j: (j, 0)),
        ],
        out_shape=[
            jax.ShapeDtypeStruct((m, n), jnp.float32),
            jax.ShapeDtypeStruct((m, n), jnp.float32),
        ],
    )(re, im, wc, ws)


def _real_fwd_body(x_ref, wc_ref, ws_ref, ore_ref, oim_ref):
    x = x_ref[...]
    ore_ref[...] = jnp.dot(x, wc_ref[...], preferred_element_type=jnp.float32)
    oim_ref[...] = -jnp.dot(x, ws_ref[...], preferred_element_type=jnp.float32)


def _real_fwd_mm(x, wc, ws, bm=128):
    m, k = x.shape
    n = wc.shape[1]
    return pl.pallas_call(
        _real_fwd_body,
        grid=(m // bm,),
        in_specs=[
            pl.BlockSpec((bm, k), lambda j: (j, 0)),
            pl.BlockSpec((k, n), lambda j: (0, 0)),
            pl.BlockSpec((k, n), lambda j: (0, 0)),
        ],
        out_specs=[
            pl.BlockSpec((bm, n), lambda j: (j, 0)),
            pl.BlockSpec((bm, n), lambda j: (j, 0)),
        ],
        out_shape=[
            jax.ShapeDtypeStruct((m, n), jnp.float32),
            jax.ShapeDtypeStruct((m, n), jnp.float32),
        ],
    )(x, wc, ws)


def _real_inv_body(re_ref, im_ref, wc_ref, ws_ref, o_ref):
    o_ref[...] = jnp.dot(re_ref[...], wc_ref[...], preferred_element_type=jnp.float32) - \
        jnp.dot(im_ref[...], ws_ref[...], preferred_element_type=jnp.float32)


def _real_inv_mm(re, im, wc, ws, bm=128):
    m, k = re.shape
    n = wc.shape[1]
    return pl.pallas_call(
        _real_inv_body,
        grid=(m // bm,),
        in_specs=[
            pl.BlockSpec((bm, k), lambda j: (j, 0)),
            pl.BlockSpec((bm, k), lambda j: (j, 0)),
            pl.BlockSpec((k, n), lambda j: (0, 0)),
            pl.BlockSpec((k, n), lambda j: (0, 0)),
        ],
        out_specs=pl.BlockSpec((bm, n), lambda j: (j, 0)),
        out_shape=jax.ShapeDtypeStruct((m, n), jnp.float32),
    )(re, im, wc, ws)


def _pointwise_body(re_ref, im_ref, w_ref, pre_ref, pim_ref):
    w = w_ref[...]
    pre_ref[...] = jnp.sum(w * im_ref[0], axis=0)[None, None]
    pim_ref[...] = -jnp.sum(w * re_ref[0], axis=0)[None, None]


def _pointwise(re, im, w, bl=51840):
    b, c, f = re.shape
    return pl.pallas_call(
        _pointwise_body,
        grid=(b, f // bl),
        in_specs=[
            pl.BlockSpec((1, c, bl), lambda i, j: (i, 0, j)),
            pl.BlockSpec((1, c, bl), lambda i, j: (i, 0, j)),
            pl.BlockSpec((c, bl), lambda i, j: (0, j)),
        ],
        out_specs=[
            pl.BlockSpec((1, 1, bl), lambda i, j: (i, 0, j)),
            pl.BlockSpec((1, 1, bl), lambda i, j: (i, 0, j)),
        ],
        out_shape=[
            jax.ShapeDtypeStruct((b, 1, f), jnp.float32),
            jax.ShapeDtypeStruct((b, 1, f), jnp.float32),
        ],
    )(re, im, w)


def _norm_body(phi_ref, off_ref, scale_ref, out_ref):
    out_ref[...] = (phi_ref[...] - off_ref[0, 0, 0]) * scale_ref[0, 0, 0]


def _rasterize(pts, vals, res):
    dim = pts.shape[-1]
    bs, npts = pts.shape[0], pts.shape[1]
    nf = vals.shape[-1]
    size = jnp.asarray((res, res, res), dtype=pts.dtype)
    cubesize = 1.0 / size
    ind0 = jnp.floor(pts / cubesize).astype(jnp.int32)
    ind1 = jnp.mod(jnp.ceil(pts / cubesize), size).astype(jnp.int32)
    xyz0 = ind0.astype(pts.dtype) * cubesize
    xyz1 = (ind0.astype(pts.dtype) + 1.0) * cubesize
    b_idx = jnp.broadcast_to(jnp.arange(bs)[:, None], (bs, npts))
    raster = jnp.zeros((bs, nf, res, res, res), dtype=vals.dtype)
    for c in itertools.product((0, 1), repeat=dim):
        idx = [ind1[..., d] if c[d] else ind0[..., d] for d in range(dim)]
        pos = jnp.stack([xyz0[..., d] if c[d] else xyz1[..., d] for d in range(dim)], axis=-1)
        w = jnp.prod(jnp.abs(pts - pos) / cubesize, axis=-1)
        contrib = w[..., None] * vals
        raster = raster.at[b_idx, :, idx[0], idx[1], idx[2]].add(contrib)
    return raster


def _interp1(grid, pts):
    bs, npts = pts.shape[0], pts.shape[1]
    size = jnp.asarray(grid.shape[1:], dtype=pts.dtype)
    cubesize = 1.0 / size
    ind0 = jnp.floor(pts / cubesize).astype(jnp.int32)
    ind1 = jnp.mod(jnp.ceil(pts / cubesize), size).astype(jnp.int32)
    xyz0 = ind0.astype(pts.dtype) * cubesize
    xyz1 = (ind0.astype(pts.dtype) + 1.0) * cubesize
    b_idx = jnp.broadcast_to(jnp.arange(bs)[:, None], (bs, npts))
    out = jnp.zeros(pts.shape[:2], dtype=grid.dtype)
    for c in itertools.product((0, 1), repeat=3):
        idx = [ind1[..., d] if c[d] else ind0[..., d] for d in range(3)]
        pos = jnp.stack([xyz0[..., d] if c[d] else xyz1[..., d] for d in range(3)], axis=-1)
        w = jnp.prod(jnp.abs(pts - pos) / cubesize, axis=-1)
        out = out + grid[b_idx, idx[0], idx[1], idx[2]] * w
    return out


def kernel(V, N):
    r = _RES
    wz_c, wz_s, wxy_c, wxy_s = [jnp.asarray(a) for a in _fwd_mats()]
    iwxy_c, iwxy_s, iwz_c, iwz_s = [jnp.asarray(a) for a in _inv_mats()]
    wpt = jnp.asarray(_pointwise_w())

    raster = _rasterize(V, N, r)

    a = raster.reshape(6 * r * r, r)
    t_re, t_im = _real_fwd_mm(a, wz_c, wz_s)
    t_re = jnp.moveaxis(t_re.reshape(6, r, r, _KZ), 2, 3).reshape(6 * r * _KZ, r)
    t_im = jnp.moveaxis(t_im.reshape(6, r, r, _KZ), 2, 3).reshape(6 * r * _KZ, r)
    u_re, u_im = _cplx_mm(t_re, t_im, wxy_c, wxy_s)
    u_re = jnp.moveaxis(u_re.reshape(6, r, _KZ, _KXY), 1, 3).reshape(6 * _KZ * _KXY, r)
    u_im = jnp.moveaxis(u_im.reshape(6, r, _KZ, _KXY), 1, 3).reshape(6 * _KZ * _KXY, r)
    s_re, s_im = _cplx_mm(u_re, u_im, wxy_c, wxy_s)

    f = _KZ * _KXY * _KXY
    s_re = s_re.reshape(2, 3, f)
    s_im = s_im.reshape(2, 3, f)
    p_re, p_im = _pointwise(s_re, s_im, wpt)

    p_re = p_re.reshape(2 * _KZ * _KXY, _KXY)
    p_im = p_im.reshape(2 * _KZ * _KXY, _KXY)
    q_re, q_im = _cplx_mm(p_re, p_im, iwxy_c, iwxy_s, bm=144)
    q_re = jnp.moveaxis(q_re.reshape(2, _KZ, _KXY, r), 2, 3).reshape(2 * _KZ * r, _KXY)
    q_im = jnp.moveaxis(q_im.reshape(2, _KZ, _KXY, r), 2, 3).reshape(2 * _KZ * r, _KXY)
    v_re, v_im = _cplx_mm(q_re, q_im, iwxy_c, iwxy_s, bm=128)
    v_re = jnp.moveaxis(v_re.reshape(2, _KZ, r, r), 1, 3).reshape(2 * r * r, _KZ)
    v_im = jnp.moveaxis(v_im.reshape(2, _KZ, r, r), 1, 3).reshape(2 * r * r, _KZ)
    phi = _real_inv_mm(v_re, v_im, iwz_c, iwz_s).reshape(2, r, r, r)

    fv = _interp1(phi, V)
    offset = jnp.mean(fv, axis=-1)
    fv0 = phi[:, 0, 0, 0] - offset
    scale = -0.5 / jnp.abs(fv0)
    off_arr = jnp.broadcast_to(offset[:, None, None], (2, 8, 128))
    scale_arr = jnp.broadcast_to(scale[:, None, None], (2, 8, 128))
    out = pl.pallas_call(
        _norm_body,
        grid=(2,),
        in_specs=[
            pl.BlockSpec((1, r, r, r), lambda b: (b, 0, 0, 0)),
            pl.BlockSpec((1, 8, 128), lambda b: (b, 0, 0)),
            pl.BlockSpec((1, 8, 128), lambda b: (b, 0, 0)),
        ],
        out_specs=pl.BlockSpec((1, r, r, r), lambda b: (b, 0, 0, 0)),
        out_shape=jax.ShapeDtypeStruct((2, r, r, r), jnp.float32),
    )(phi, off_arr, scale_arr)
    return out
